# Initial kernel scaffold; baseline (speedup 1.0000x reference)
import jax
import jax.numpy as jnp
from jax.experimental import pallas as pl


def kernel(x, edge_index, meta, batch, emb_W, emb_b, conv_Wh, conv_bh, conv_Wo, conv_bo, hid_W, hid_b, out_W, out_b):
    raise NotImplementedError("write your pallas kernel here")


# speedup vs baseline: 2.1449x; 2.1449x over previous
import jax
import jax.numpy as jnp
from jax import lax
from jax.experimental import pallas as pl
from jax.experimental.pallas import tpu as pltpu
from jax.experimental.pallas import tpu_sc as plsc

N = 50000
E = 800000
G = 64
CH = 64
HALF = 32
CONVS = 8
GHID = 256
HIDDEN = 1024

ROWS = 400
NBLK = N // ROWS

TILES = 16
E_PER_TILE = E // TILES
EDGE_K = 80
N_CHUNKS = E_PER_TILE // EDGE_K
R_PER_TILE = 3128


def _segsum_body(xsplit, src2, dst2, zeros, y, acc, gidx, sidx, rows, sem):
    c = lax.axis_index("c")
    s = lax.axis_index("s")
    ebase = s * E_PER_TILE
    rbase = jnp.minimum(s * R_PER_TILE, N - R_PER_TILE)
    for d in range(2):
        pltpu.sync_copy(zeros, acc.at[pl.ds(rbase, R_PER_TILE), :])
        plsc.subcore_barrier()

        def chunk(j, carry):
            off = ebase + j * EDGE_K
            if d == 0:
                pltpu.sync_copy(src2.at[pl.ds(c * E + off, EDGE_K)], gidx)
                pltpu.sync_copy(dst2.at[pl.ds(off, EDGE_K)], sidx)
            else:
                pltpu.sync_copy(dst2.at[pl.ds(c * E + off, EDGE_K)], gidx)
                pltpu.sync_copy(src2.at[pl.ds(off, EDGE_K)], sidx)
            pltpu.async_copy(xsplit.at[gidx], rows, sem).wait()
            pltpu.sync_copy(rows, acc.at[sidx], add=True)
            return carry

        lax.fori_loop(0, N_CHUNKS, chunk, 0)
        plsc.subcore_barrier()
        pltpu.sync_copy(acc.at[pl.ds(rbase, R_PER_TILE), :],
                        y.at[d, c, pl.ds(rbase, R_PER_TILE), :])
        plsc.subcore_barrier()


def _segsum(xsplit, src2, dst2, zeros):
    mesh = plsc.VectorSubcoreMesh(core_axis_name="c", subcore_axis_name="s")
    return pl.kernel(
        _segsum_body,
        out_type=jax.ShapeDtypeStruct((2, 2, N, HALF), jnp.float32),
        mesh=mesh,
        scratch_types=[
            pltpu.VMEM_SHARED((N, HALF), jnp.float32),
            pltpu.VMEM((EDGE_K,), jnp.int32),
            pltpu.VMEM((EDGE_K,), jnp.int32),
            pltpu.VMEM((EDGE_K, HALF), jnp.float32),
            pltpu.SemaphoreType.DMA,
        ],
        compiler_params=pltpu.CompilerParams(use_tc_tiling_on_sc=False),
    )(xsplit, src2, dst2, zeros)


def _mlp_body(y_ref, wh_ref, bh_ref, wo_ref, bo_ref, h2_ref, st_ref,
              asum, asq):
    i = pl.program_id(0)
    h = jnp.concatenate(
        [y_ref[0, 0], y_ref[0, 1], y_ref[1, 0], y_ref[1, 1]], axis=1)
    h1 = jnp.dot(h, wh_ref[...], preferred_element_type=jnp.float32)
    h1 = jnp.maximum(h1 + bh_ref[...], 0.0)
    h2 = jnp.dot(h1, wo_ref[...], preferred_element_type=jnp.float32)
    h2 = h2 + bo_ref[...]
    h2_ref[...] = h2
    ps = jnp.sum(h2, axis=0, keepdims=True)
    pq = jnp.sum(h2 * h2, axis=0, keepdims=True)

    @pl.when(i == 0)
    def _():
        asum[...] = ps
        asq[...] = pq

    @pl.when(i > 0)
    def _():
        asum[...] += ps
        asq[...] += pq

    @pl.when(i == NBLK - 1)
    def _():
        st_ref[0:1] = asum[...]
        st_ref[1:2] = asq[...]


def _mlp(y, wh, bh, wo, bo):
    return pl.pallas_call(
        _mlp_body,
        grid=(NBLK,),
        in_specs=[
            pl.BlockSpec((2, 2, ROWS, HALF), lambda i: (0, 0, i, 0)),
            pl.BlockSpec((2 * CH, GHID), lambda i: (0, 0)),
            pl.BlockSpec((1, GHID), lambda i: (0, 0)),
            pl.BlockSpec((GHID, CH), lambda i: (0, 0)),
            pl.BlockSpec((1, CH), lambda i: (0, 0)),
        ],
        out_specs=[
            pl.BlockSpec((ROWS, CH), lambda i: (i, 0)),
            pl.BlockSpec((2, CH), lambda i: (0, 0)),
        ],
        out_shape=[
            jax.ShapeDtypeStruct((N, CH), jnp.float32),
            jax.ShapeDtypeStruct((2, CH), jnp.float32),
        ],
        scratch_shapes=[
            pltpu.VMEM((1, CH), jnp.float32),
            pltpu.VMEM((1, CH), jnp.float32),
        ],
    )(y, wh, bh, wo, bo)


def _norm2_body(h2_ref, xp_ref, st_ref, xn_ref, xs_ref):
    mean = st_ref[0:1]
    inv = 1.0 / jnp.sqrt(st_ref[1:2] + 1e-5)
    v = jnp.maximum(xp_ref[...] + (h2_ref[...] - mean) * inv, 0.0)
    xn_ref[...] = v
    xs_ref[0] = v[:, :HALF]
    xs_ref[1] = v[:, HALF:]


def _norm2(h2, xp, st):
    return pl.pallas_call(
        _norm2_body,
        grid=(NBLK,),
        in_specs=[
            pl.BlockSpec((ROWS, CH), lambda i: (i, 0)),
            pl.BlockSpec((ROWS, CH), lambda i: (i, 0)),
            pl.BlockSpec((2, CH), lambda i: (0, 0)),
        ],
        out_specs=[
            pl.BlockSpec((ROWS, CH), lambda i: (i, 0)),
            pl.BlockSpec((2, ROWS, HALF), lambda i: (0, i, 0)),
        ],
        out_shape=[
            jax.ShapeDtypeStruct((N, CH), jnp.float32),
            jax.ShapeDtypeStruct((2, N, HALF), jnp.float32),
        ],
    )(h2, xp, st)


POOL_CHUNK = 256
FEAT = (CONVS + 1) * CH


def _lower_bound(bvm, val):
    def step(_, lohi):
        lo, hi = lohi
        mid = (lo + hi) // 2
        midv = jnp.full((16,), mid, jnp.int32)
        bmid = plsc.load_gather(bvm, [midv])[0]
        go_hi = bmid < val
        return (jnp.where(go_hi, mid + 1, lo), jnp.where(go_hi, hi, mid))

    lo, _ = lax.fori_loop(0, 16, step, (jnp.int32(0), jnp.int32(N)))
    return lo


def _pool_body(batch, *rest):
    xs = rest[:CONVS + 1]
    pooled = rest[CONVS + 1]
    bvm, chunk, acc, sem = rest[CONVS + 2:]
    c = lax.axis_index("c")
    s = lax.axis_index("s")
    wid = c * TILES + s
    pltpu.sync_copy(batch, bvm)
    for gg in range(2):
        g = wid * 2 + gg
        s0 = _lower_bound(bvm, g)
        e0 = _lower_bound(bvm, g + 1)
        s0a = (s0 // 8) * 8
        for j in range(FEAT // 16):
            acc[pl.ds(j * 16, 16)] = jnp.full((16,), -jnp.inf, jnp.float32)
        nch = (e0 - s0a + POOL_CHUNK - 1) // POOL_CHUNK
        for k in range(CONVS + 1):
            def ck(jc, carry):
                base = s0a + jc * POOL_CHUNK
                base_c = jnp.minimum(base, N - POOL_CHUNK)
                pltpu.sync_copy(xs[k].at[pl.ds(base_c, POOL_CHUNK), :], chunk)

                def row(r, carry2):
                    gi = base_c + r
                    giv = jnp.full((16,), gi, jnp.int32)
                    ok = ((giv >= jnp.full((16,), s0, jnp.int32))
                          & (giv < jnp.full((16,), e0, jnp.int32)))
                    ninf = jnp.full((16,), -jnp.inf, jnp.float32)
                    for jj in range(CH // 16):
                        sl = pl.ds(k * CH + jj * 16, 16)
                        val = jnp.where(ok, chunk[r, pl.ds(jj * 16, 16)],
                                        ninf)
                        acc[sl] = jnp.maximum(acc[sl], val)
                    return carry2

                lax.fori_loop(0, POOL_CHUNK, row, 0)
                return carry

            lax.fori_loop(0, nch, ck, 0)
        pltpu.sync_copy(acc, pooled.at[pl.ds(g * FEAT, FEAT)])


def _pool(batch, xs_list):
    mesh = plsc.VectorSubcoreMesh(core_axis_name="c", subcore_axis_name="s")
    return pl.kernel(
        _pool_body,
        out_type=jax.ShapeDtypeStruct((G * FEAT,), jnp.float32),
        mesh=mesh,
        scratch_types=[
            pltpu.VMEM((N,), jnp.int32),
            pltpu.VMEM((POOL_CHUNK, CH), jnp.float32),
            pltpu.VMEM((FEAT,), jnp.float32),
            pltpu.SemaphoreType.DMA,
        ],
        compiler_params=pltpu.CompilerParams(
            use_tc_tiling_on_sc=False, needs_layout_passes=False),
    )(batch, *xs_list)


def _head_body(p_ref, m_ref, w1_ref, w2_ref, hb_ref, ow_ref, ob_ref, o_ref):
    z = jnp.dot(p_ref[...], w1_ref[...], preferred_element_type=jnp.float32,
                 precision=lax.Precision.HIGHEST)
    z = z + jnp.dot(m_ref[...], w2_ref[...], preferred_element_type=jnp.float32,
                 precision=lax.Precision.HIGHEST)
    z = jnp.maximum(z + hb_ref[...], 0.0)
    o = jnp.dot(z, ow_ref[...], preferred_element_type=jnp.float32,
                 precision=lax.Precision.HIGHEST)
    o_ref[...] = o + ob_ref[...]


def _head(pooled, metap, w1, w2, hb, owp, obp):
    return pl.pallas_call(
        _head_body,
        out_shape=jax.ShapeDtypeStruct((G, 8), jnp.float32),
    )(pooled, metap, w1, w2, hb, owp, obp)


def kernel(x, edge_index, meta, batch, emb_W, emb_b, conv_Wh, conv_bh,
           conv_Wo, conv_bo, hid_W, hid_b, out_W, out_b):
    f32 = jnp.float32
    src = edge_index[0]
    dst = edge_index[1]
    src2 = jnp.concatenate([src, src + N])
    dst2 = jnp.concatenate([dst, dst + N])
    zeros = jnp.zeros((R_PER_TILE, HALF), f32)

    xcur = x @ emb_W + emb_b
    xsplit = jnp.concatenate([xcur[:, :HALF], xcur[:, HALF:]], axis=0)
    xs_list = [xcur]
    for i in range(CONVS):
        y = _segsum(xsplit, src2, dst2, zeros)
        h2, st = _mlp(y, conv_Wh[i], conv_bh[i].reshape(1, GHID),
                      conv_Wo[i], conv_bo[i].reshape(1, CH))
        stj = jnp.concatenate([jnp.mean(h2, axis=0, keepdims=True),
                               jnp.var(h2, axis=0, keepdims=True)])
        xcur, xsp = _norm2(h2, xcur, stj)
        xsplit = xsp.reshape(2 * N, HALF)
        xs_list.append(xcur)

    pooled = _pool(batch, xs_list).reshape(G, FEAT)
    metap = jnp.pad(meta, ((0, 0), (0, 1)))
    w1 = hid_W[:FEAT]
    w2 = jnp.pad(hid_W[FEAT:], ((0, 1), (0, 0)))
    owp = jnp.pad(out_W, ((0, 0), (0, 7)))
    obp = jnp.pad(out_b.reshape(1, 1), ((0, 0), (0, 7)))
    out = _head(pooled, metap, w1, w2, hid_b.reshape(1, HIDDEN), owp, obp)
    return out[:, 0]
